# baseline (speedup 1.0000x reference)
import functools

import jax
import jax.numpy as jnp
from jax import lax
from jax.experimental import pallas as pl
from jax.experimental.pallas import tpu as pltpu
from jax.experimental.pallas import tpu_sc as plsc

_WORD_NUM = 1000000
_EMBED_DIM = 64
_PAD_DIM = 128
_BATCH = 16384
_HIST = 50

_info = plsc.get_sparse_core_info()
_NC = _info.num_cores
_NS = _info.num_subcores
_NW = _NC * _NS
_BPW = _BATCH // _NW
_CB = 128
_SPLIT = _BPW // _CB
_G = _HIST * _SPLIT
_NBUF = 4

_mesh = plsc.VectorSubcoreMesh(core_axis_name="c", subcore_axis_name="s")


@functools.partial(
    pl.kernel,
    mesh=_mesh,
    out_type=jax.ShapeDtypeStruct((_HIST, _BATCH, _EMBED_DIM), jnp.float32),
    compiler_params=pltpu.CompilerParams(use_tc_tiling_on_sc=False),
    scratch_types=(
        [pltpu.VMEM((_HIST, _BPW), jnp.int32),
         pltpu.VMEM((_NBUF, _CB, _PAD_DIM), jnp.float32)]
        + [pltpu.SemaphoreType.DMA] * (2 * _NBUF)
    ),
)
def _sc_gather(table_hbm, idx_hbm, out_hbm, idx_v, rows_v, *sems):
    sem_g = sems[:_NBUF]
    sem_o = sems[_NBUF:]
    wid = lax.axis_index("s") * _NC + lax.axis_index("c")
    base = wid * _BPW
    pltpu.sync_copy(idx_hbm.at[:, pl.ds(base, _BPW)], idx_v)

    def chunk_idx(g):
        h = g // _SPLIT
        r = g % _SPLIT
        return idx_v.at[h, pl.ds(r * _CB, _CB)]

    def start_gather(g, b):
        pltpu.async_copy(table_hbm.at[chunk_idx(g)], rows_v.at[b], sem_g[b])

    def wait_gather(g, b):
        pltpu.make_async_copy(table_hbm.at[chunk_idx(g)], rows_v.at[b],
                              sem_g[b]).wait()

    def valid_rows(b):
        return rows_v.at[b, :, pl.ds(0, _EMBED_DIM)]

    def out_slice(g):
        h = g // _SPLIT
        r = g % _SPLIT
        return out_hbm.at[h, pl.ds(base + r * _CB, _CB)]

    for b in range(_NBUF):
        start_gather(b, b)

    def super_body(s, carry):
        for b in range(_NBUF):
            g = s * _NBUF + b
            wait_gather(g, b)
            pltpu.async_copy(valid_rows(b), out_slice(g), sem_o[b])

            @pl.when(g + _NBUF < _G)
            def _():
                pltpu.make_async_copy(valid_rows(b), out_slice(g),
                                      sem_o[b]).wait()
                start_gather(g + _NBUF, b)

        return carry

    lax.fori_loop(0, _G // _NBUF, super_body, 0)

    for b in range(_NBUF):
        g = _G - _NBUF + b
        pltpu.make_async_copy(valid_rows(b), out_slice(g), sem_o[b]).wait()


def kernel(input_x, weight):
    idx_t = jnp.transpose(input_x.astype(jnp.int32))
    w_pad = jnp.pad(weight, ((0, 0), (0, _PAD_DIM - _EMBED_DIM)))
    out_t = _sc_gather(w_pad, idx_t)
    return jnp.transpose(out_t, (1, 0, 2))


# speedup vs baseline: 1.0083x; 1.0083x over previous
import functools

import jax
import jax.numpy as jnp
from jax import lax
from jax.experimental import pallas as pl
from jax.experimental.pallas import tpu as pltpu
from jax.experimental.pallas import tpu_sc as plsc

_WORD_NUM = 1000000
_EMBED_DIM = 64
_BATCH = 16384
_HIST = 50

_info = plsc.get_sparse_core_info()
_NC = _info.num_cores
_NS = _info.num_subcores
_NW = _NC * _NS
_BPW = _BATCH // _NW
_CB = 128
_SPLIT = _BPW // _CB
_G = _HIST * _SPLIT
_NBUF = 8

_mesh = plsc.VectorSubcoreMesh(core_axis_name="c", subcore_axis_name="s")


@functools.partial(
    pl.kernel,
    mesh=_mesh,
    out_type=jax.ShapeDtypeStruct((_HIST, _BATCH, _EMBED_DIM), jnp.float32),
    compiler_params=pltpu.CompilerParams(use_tc_tiling_on_sc=False),
    scratch_types=(
        [pltpu.VMEM((_HIST, _BPW), jnp.int32),
         pltpu.VMEM((_NBUF, _CB, _EMBED_DIM), jnp.float32)]
        + [pltpu.SemaphoreType.DMA] * (2 * _NBUF)
    ),
)
def _sc_gather(table_hbm, idx_hbm, out_hbm, idx_v, rows_v, *sems):
    sem_g = sems[:_NBUF]
    sem_o = sems[_NBUF:]
    wid = lax.axis_index("s") * _NC + lax.axis_index("c")
    base = wid * _BPW
    pltpu.sync_copy(idx_hbm.at[:, pl.ds(base, _BPW)], idx_v)

    def chunk_idx(g):
        h = g // _SPLIT
        r = g % _SPLIT
        return idx_v.at[h, pl.ds(r * _CB, _CB)]

    def start_gather(g, b):
        pltpu.async_copy(table_hbm.at[chunk_idx(g)], rows_v.at[b], sem_g[b])

    def wait_gather(g, b):
        pltpu.make_async_copy(table_hbm.at[chunk_idx(g)], rows_v.at[b],
                              sem_g[b]).wait()

    def out_slice(g):
        h = g // _SPLIT
        r = g % _SPLIT
        return out_hbm.at[h, pl.ds(base + r * _CB, _CB)]

    for b in range(_NBUF):
        start_gather(b, b)

    def super_body(s, carry):
        for b in range(_NBUF):
            g = s * _NBUF + b
            wait_gather(g, b)
            pltpu.async_copy(rows_v.at[b], out_slice(g), sem_o[b])

            @pl.when(g + _NBUF < _G)
            def _():
                pltpu.make_async_copy(rows_v.at[b], out_slice(g),
                                      sem_o[b]).wait()
                start_gather(g + _NBUF, b)

        return carry

    lax.fori_loop(0, _G // _NBUF, super_body, 0)

    for b in range(_NBUF):
        g = _G - _NBUF + b
        pltpu.make_async_copy(rows_v.at[b], out_slice(g), sem_o[b]).wait()


def kernel(input_x, weight):
    idx_t = jnp.transpose(input_x.astype(jnp.int32))
    out_t = _sc_gather(weight, idx_t)
    return jnp.transpose(out_t, (1, 0, 2))


# speedup vs baseline: 1.0094x; 1.0011x over previous
import functools

import jax
import jax.numpy as jnp
from jax import lax
from jax.experimental import pallas as pl
from jax.experimental.pallas import tpu as pltpu
from jax.experimental.pallas import tpu_sc as plsc

_WORD_NUM = 1000000
_EMBED_DIM = 64
_BATCH = 16384
_HIST = 50

_info = plsc.get_sparse_core_info()
_NC = _info.num_cores
_NS = _info.num_subcores
_NW = _NC * _NS
_BPW = _BATCH // _NW
_CB = 256
_SPLIT = _BPW // _CB
_G = _HIST * _SPLIT
_NBUF = 4

_mesh = plsc.VectorSubcoreMesh(core_axis_name="c", subcore_axis_name="s")


@functools.partial(
    pl.kernel,
    mesh=_mesh,
    out_type=jax.ShapeDtypeStruct((_HIST, _BATCH, _EMBED_DIM), jnp.float32),
    compiler_params=pltpu.CompilerParams(use_tc_tiling_on_sc=False),
    scratch_types=(
        [pltpu.VMEM((_HIST, _BPW), jnp.int32),
         pltpu.VMEM((_NBUF, _CB, _EMBED_DIM), jnp.float32)]
        + [pltpu.SemaphoreType.DMA] * (2 * _NBUF)
    ),
)
def _sc_gather(table_hbm, idx_hbm, out_hbm, idx_v, rows_v, *sems):
    sem_g = sems[:_NBUF]
    sem_o = sems[_NBUF:]
    wid = lax.axis_index("s") * _NC + lax.axis_index("c")
    base = wid * _BPW
    pltpu.sync_copy(idx_hbm.at[:, pl.ds(base, _BPW)], idx_v)

    def chunk_idx(g):
        h = g // _SPLIT
        r = g % _SPLIT
        return idx_v.at[h, pl.ds(r * _CB, _CB)]

    def start_gather(g, b):
        pltpu.async_copy(table_hbm.at[chunk_idx(g)], rows_v.at[b], sem_g[b])

    def wait_gather(g, b):
        pltpu.make_async_copy(table_hbm.at[chunk_idx(g)], rows_v.at[b],
                              sem_g[b]).wait()

    def out_slice(g):
        h = g // _SPLIT
        r = g % _SPLIT
        return out_hbm.at[h, pl.ds(base + r * _CB, _CB)]

    for b in range(_NBUF):
        start_gather(b, b)

    def super_body(s, carry):
        for b in range(_NBUF):
            g = s * _NBUF + b
            wait_gather(g, b)
            pltpu.async_copy(rows_v.at[b], out_slice(g), sem_o[b])

            @pl.when(g + _NBUF < _G)
            def _():
                pltpu.make_async_copy(rows_v.at[b], out_slice(g),
                                      sem_o[b]).wait()
                start_gather(g + _NBUF, b)

        return carry

    lax.fori_loop(0, _G // _NBUF, super_body, 0)

    for b in range(_NBUF):
        g = _G - _NBUF + b
        pltpu.make_async_copy(rows_v.at[b], out_slice(g), sem_o[b]).wait()


def kernel(input_x, weight):
    idx_t = jnp.transpose(input_x.astype(jnp.int32))
    out_t = _sc_gather(weight, idx_t)
    return jnp.transpose(out_t, (1, 0, 2))
